# baseline (speedup 1.0000x reference)
import jax
import jax.numpy as jnp
from jax.experimental import pallas as pl


_CHUNK = 256


def _split_hi_lo(v):
    hi = v.astype(jnp.bfloat16)
    lo = (v - hi.astype(jnp.float32)).astype(jnp.bfloat16)
    return hi, lo


def _chamfer_kernel(x_ref, y_ref, out_x_ref, out_y_ref):
    d, n = x_ref.shape
    n_chunks = n // _CHUNK

    xv = x_ref[...]
    yv = y_ref[...]
    nx = jnp.sum(xv * xv, axis=0, keepdims=True)
    at_full = jnp.concatenate(
        [-2.0 * xv, nx, jnp.ones((1, n), jnp.float32)], axis=0
    )
    ny = jnp.sum(yv * yv, axis=0, keepdims=True)
    bt_full = jnp.concatenate(
        [yv, jnp.ones((1, n), jnp.float32), ny], axis=0
    )

    a_hi, a_lo = _split_hi_lo(at_full)
    b_hi, b_lo = _split_hi_lo(bt_full)
    aat = jnp.concatenate(
        [
            a_hi,
            a_hi[0:d, :],
            a_hi[d + 1 : d + 2, :],
            a_lo[0 : d + 1, :],
            a_lo[0:d, :],
        ],
        axis=0,
    )
    bbt = jnp.concatenate(
        [
            b_hi,
            b_lo[0:d, :],
            b_lo[d + 1 : d + 2, :],
            b_hi[0 : d + 1, :],
            b_lo[0:d, :],
        ],
        axis=0,
    )

    def chunk_dist(i):
        a_chunk = aat[:, i * _CHUNK : (i + 1) * _CHUNK]
        return jax.lax.dot_general(
            a_chunk,
            bbt,
            (((0,), (0,)), ((), ())),
            preferred_element_type=jnp.float32,
        )

    ymin = jnp.full((1, n), jnp.inf, dtype=jnp.float32)
    mnacc = jnp.zeros((_CHUNK, 1), dtype=jnp.float32)
    t_cur = chunk_dist(0)
    for i in range(n_chunks):
        t_next = chunk_dist(i + 1) if i + 1 < n_chunks else None
        mnacc = mnacc + jnp.min(t_cur, axis=1, keepdims=True)
        ymin = jnp.minimum(ymin, jnp.min(t_cur, axis=0, keepdims=True))
        t_cur = t_next
    out_x_ref[...] = jnp.sum(mnacc, keepdims=True)
    out_y_ref[...] = jnp.sum(ymin, keepdims=True)


def kernel(x, y):
    b, d, n = x.shape
    f32 = jnp.float32

    out_x, out_y = pl.pallas_call(
        _chamfer_kernel,
        grid=(b,),
        in_specs=[
            pl.BlockSpec((None, d, n), lambda i: (i, 0, 0)),
            pl.BlockSpec((None, d, n), lambda i: (i, 0, 0)),
        ],
        out_specs=[
            pl.BlockSpec((None, 1, 1), lambda i: (i, 0, 0)),
            pl.BlockSpec((None, 1, 1), lambda i: (i, 0, 0)),
        ],
        out_shape=[
            jax.ShapeDtypeStruct((b, 1, 1), f32),
            jax.ShapeDtypeStruct((b, 1, 1), f32),
        ],
    )(x, y)

    return (jnp.sum(out_x) + jnp.sum(out_y)) / (b * n)


# speedup vs baseline: 1.0011x; 1.0011x over previous
import jax
import jax.numpy as jnp
from jax.experimental import pallas as pl


_CHUNK = 1024


def _split_hi_lo(v):
    hi = v.astype(jnp.bfloat16)
    lo = (v - hi.astype(jnp.float32)).astype(jnp.bfloat16)
    return hi, lo


def _chamfer_kernel(x_ref, y_ref, out_x_ref, out_y_ref):
    d, n = x_ref.shape
    n_chunks = n // _CHUNK

    xv = x_ref[...]
    yv = y_ref[...]
    nx = jnp.sum(xv * xv, axis=0, keepdims=True)
    at_full = jnp.concatenate(
        [-2.0 * xv, nx, jnp.ones((1, n), jnp.float32)], axis=0
    )
    ny = jnp.sum(yv * yv, axis=0, keepdims=True)
    bt_full = jnp.concatenate(
        [yv, jnp.ones((1, n), jnp.float32), ny], axis=0
    )

    a_hi, a_lo = _split_hi_lo(at_full)
    b_hi, b_lo = _split_hi_lo(bt_full)
    aat = jnp.concatenate(
        [
            a_hi,
            a_hi[0:d, :],
            a_hi[d + 1 : d + 2, :],
            a_lo[0 : d + 1, :],
            a_lo[0:d, :],
        ],
        axis=0,
    )
    bbt = jnp.concatenate(
        [
            b_hi,
            b_lo[0:d, :],
            b_lo[d + 1 : d + 2, :],
            b_hi[0 : d + 1, :],
            b_lo[0:d, :],
        ],
        axis=0,
    )

    def chunk_dist(i):
        a_chunk = aat[:, i * _CHUNK : (i + 1) * _CHUNK]
        return jax.lax.dot_general(
            a_chunk,
            bbt,
            (((0,), (0,)), ((), ())),
            preferred_element_type=jnp.float32,
        )

    ymin = jnp.full((1, n), jnp.inf, dtype=jnp.float32)
    mnacc = jnp.zeros((_CHUNK, 1), dtype=jnp.float32)
    t_cur = chunk_dist(0)
    for i in range(n_chunks):
        t_next = chunk_dist(i + 1) if i + 1 < n_chunks else None
        mnacc = mnacc + jnp.min(t_cur, axis=1, keepdims=True)
        ymin = jnp.minimum(ymin, jnp.min(t_cur, axis=0, keepdims=True))
        t_cur = t_next
    out_x_ref[...] = jnp.sum(mnacc, keepdims=True)
    out_y_ref[...] = jnp.sum(ymin, keepdims=True)


def kernel(x, y):
    b, d, n = x.shape
    f32 = jnp.float32

    out_x, out_y = pl.pallas_call(
        _chamfer_kernel,
        grid=(b,),
        in_specs=[
            pl.BlockSpec((None, d, n), lambda i: (i, 0, 0)),
            pl.BlockSpec((None, d, n), lambda i: (i, 0, 0)),
        ],
        out_specs=[
            pl.BlockSpec((None, 1, 1), lambda i: (i, 0, 0)),
            pl.BlockSpec((None, 1, 1), lambda i: (i, 0, 0)),
        ],
        out_shape=[
            jax.ShapeDtypeStruct((b, 1, 1), f32),
            jax.ShapeDtypeStruct((b, 1, 1), f32),
        ],
    )(x, y)

    return (jnp.sum(out_x) + jnp.sum(out_y)) / (b * n)
